# baseline (speedup 1.0000x reference)
import functools

import jax
import jax.numpy as jnp
import numpy as np
from jax import lax
from jax.experimental import pallas as pl
from jax.experimental.pallas import tpu as pltpu
from jax.experimental.pallas import tpu_sc as plsc

N_NODES = 10000
N_EDGES = 160000
S = 256
V = 64
EC = 64
NET = 4
R_MAX = 10.0
NUM_GAUSS = EC - NET
D_T = 256
D_M = 512

N_PAD = 10240
NW = 32
CH = 128
NB = 1000
EB = 512
E_PAD2 = 160256
PARTS = (40960, 40960, 39936, 38400)

_SLOPE = 0.2


def _leaky(x):
    return jnp.where(x >= 0, x, 0.01 * x)


def _gv_planes(sca, vx, vy, vz, wv, ws, gb, outs):
    f32 = jnp.float32
    B = sca.shape[0]
    vs = jnp.concatenate([vx, vy, vz], axis=0)
    ys = jnp.dot(vs, wv, preferred_element_type=f32)
    vi = [ys[c * B:(c + 1) * B, :64] for c in range(3)]
    ovp = [ys[c * B:(c + 1) * B, 64:] for c in range(3)]
    vnorm = jnp.sqrt(vi[0] * vi[0] + vi[1] * vi[1] + vi[2] * vi[2])
    sc = jnp.concatenate([vnorm, sca], axis=-1)
    so = jnp.dot(sc, ws, preferred_element_type=f32)
    out_s = so[:, :outs]
    gate = jax.nn.sigmoid(so[:, outs:] + gb)
    ov = [gate * v for v in ovp]
    return out_s, ov


def _node_table_body(nh, nvx, nvy, nvz, wv, ws, gb, out):
    f32 = jnp.float32
    i32 = jnp.int32
    ns, nv = _gv_planes(nh[...], nvx[...], nvy[...], nvz[...],
                        wv[...], ws[...], gb[...], S)
    z = jnp.zeros((ns.shape[0], 64), f32)
    hi = jnp.concatenate([nv[0], nv[1], nv[2], z], axis=-1)
    lob = jax.lax.bitcast_convert_type(ns.astype(jnp.bfloat16).astype(f32), i32)
    hib = jax.lax.bitcast_convert_type(hi.astype(jnp.bfloat16).astype(f32), i32)
    packed = jax.lax.shift_right_logical(lob, 16) | (hib & i32(-65536))
    out[...] = jax.lax.bitcast_convert_type(packed, f32)


def _node_post_body(nh, nvx, nvy, nvz, agg0, agg1, agg2, agg3,
                    c_wv, c_ws, c_gb,
                    lnsg, lnsb, lnvg, lnvb, actwt,
                    o_wv, o_ws, o_gb,
                    out_s, out_v):
    f32 = jnp.float32
    cs, cv = _gv_planes(nh[...], nvx[...], nvy[...], nvz[...],
                        c_wv[...], c_ws[...], c_gb[...], S)
    a = (agg0[...] + agg1[...]) + (agg2[...] + agg3[...])
    osca = cs + a[:, :S]
    ov = [cv[c] + a[:, S + 64 * c:S + 64 * (c + 1)] for c in range(3)]
    mu = jnp.mean(osca, axis=-1, keepdims=True)
    var = jnp.mean(osca * osca, axis=-1, keepdims=True) - mu * mu
    osca = (osca - mu) * jax.lax.rsqrt(var + 1e-5) * lnsg[...] + lnsb[...]
    sv = ov[0] + ov[1] + ov[2]
    mu_v = jnp.sum(sv, axis=-1, keepdims=True) * (1.0 / 192.0)
    sq = ov[0] * ov[0] + ov[1] * ov[1] + ov[2] * ov[2]
    var_v = jnp.sum(sq, axis=-1, keepdims=True) * (1.0 / 192.0) - mu_v * mu_v
    inv = jax.lax.rsqrt(var_v + 1e-5)
    g = lnvg[...]
    b = lnvb[...]
    ov = [(ov[c] - mu_v) * inv * g[c:c + 1, :] + b[c:c + 1, :] for c in range(3)]
    s_act = _leaky(osca)
    dv = [jnp.dot(v, actwt[...], preferred_element_type=f32) for v in ov]
    dot = ov[0] * dv[0] + ov[1] * dv[1] + ov[2] * dv[2]
    d2 = dv[0] * dv[0] + dv[1] * dv[1] + dv[2] * dv[2]
    msk = (dot >= 0).astype(f32)
    ratio = (1.0 - msk) * (dot / (d2 + 1e-6))
    v_act = [_SLOPE * ov[c] + (1.0 - _SLOPE) * (ov[c] - ratio * dv[c])
             for c in range(3)]
    fs, fv = _gv_planes(s_act, v_act[0], v_act[1], v_act[2],
                        o_wv[...], o_ws[...], o_gb[...], S)
    out_s[...] = fs
    out_v[...] = jnp.concatenate([fv[0], fv[1], fv[2]], axis=-1)


def _edge_mlp_body(g, ea_t, ev_t,
                   w1a, w2, offs, e_cmb, e_gb, actwt,
                   se_cmb, scab, e2nb, n2ewt, n2eb, evnwt,
                   m_wv, m_ws, m_gb,
                   out, *, base=None):
    f32 = jnp.float32
    i32 = jnp.int32
    gi = jax.lax.bitcast_convert_type(g[...], i32)
    ns_j = jax.lax.bitcast_convert_type(jax.lax.shift_left(gi, 16), f32)
    hiv = jax.lax.bitcast_convert_type(gi & i32(-65536), f32)
    eye = (jax.lax.broadcasted_iota(jnp.int32, (EB, EB), 0)
           == jax.lax.broadcasted_iota(jnp.int32, (EB, EB), 1)).astype(f32)
    tdn = (((1,), (1,)), ((), ()))
    evec = jax.lax.dot_general(eye, ev_t[...], tdn,
                               preferred_element_type=f32)
    eattr = jax.lax.dot_general(eye, ea_t[...], tdn,
                                preferred_element_type=f32)
    d2r = jnp.sum(evec * evec, axis=-1, keepdims=True)
    d = jnp.sqrt(d2r)
    diff = d - offs[...]
    step = R_MAX / (NUM_GAUSS - 1)
    coeff = -0.5 / (step * step)
    gexp = jnp.exp(coeff * diff * diff)
    esf = jnp.concatenate([gexp, eattr], axis=-1)
    inv_d = 1.0 / (d + 1e-7)
    evn = evec * inv_d
    q = d2r * inv_d * inv_d
    vnorm_e = jnp.sqrt(q) * jnp.abs(w1a[...])
    sc = jnp.concatenate([vnorm_e, esf], axis=-1)
    er = jnp.dot(sc, e_cmb[...], preferred_element_type=f32)
    es0 = er[:, :EC]
    gate_e = jax.nn.sigmoid(er[:, EC:] + e_gb[...])
    a = gate_e * w2[...]
    A = jnp.dot(a, actwt[...], preferred_element_type=f32)
    dot = a * A * q
    dd = A * A * q
    msk = (dot >= 0).astype(f32)
    a2 = _SLOPE * a + (1.0 - _SLOPE) * (a - (1.0 - msk) * (dot / (dd + 1e-6)) * A)
    es = _leaky(es0)
    ser = jnp.dot(es, se_cmb[...], preferred_element_type=f32)
    y_sca = ns_j * (ser[:, :S] + scab[...])
    alpha = ser[:, S:] + e2nb[...]
    beta = (jnp.dot(ns_j, n2ewt[...], preferred_element_type=f32) + n2eb[...]) * \
        jnp.dot(a2, evnwt[...], preferred_element_type=f32)
    yv = [alpha * hiv[:, 64 * c:64 * (c + 1)] + beta * evn[:, c:c + 1]
          for c in range(3)]
    m_sca, m_vec = _gv_planes(y_sca, yv[0], yv[1], yv[2],
                              m_wv[...], m_ws[...], m_gb[...], S)
    C = 0.5 * (jnp.cos(d * (np.pi / R_MAX)) + 1.0)
    C = C * (d <= R_MAX).astype(f32) * (d >= 0.0).astype(f32)
    if base is not None:
        eid = base + pl.program_id(0) * EB + \
            jax.lax.broadcasted_iota(jnp.int32, (EB, 1), 0)
        C = C * (eid < N_EDGES).astype(f32)
    z = jnp.zeros((EB, 64), f32)
    out[...] = jnp.concatenate(
        [m_sca * C, m_vec[0] * C, m_vec[1] * C, m_vec[2] * C, z], axis=-1)


def _sc_mesh():
    return plsc.VectorSubcoreMesh(core_axis_name="c", subcore_axis_name="s",
                                  num_cores=2)


def _pick_ch(per, cap, align):
    for d in range(cap - cap % align, align - 1, -align):
        if per % d == 0:
            return d
    raise ValueError(per)


def _sc_gather(table, idx3):
    nw, nch, ch = idx3.shape
    per_w = nch * ch
    n = NW * per_w

    @functools.partial(
        pl.kernel, mesh=_sc_mesh(),
        out_type=jax.ShapeDtypeStruct((n, D_T), jnp.float32),
        scratch_types=[
            pltpu.VMEM((nch, ch), jnp.int32),
            pltpu.VMEM((ch, D_T), jnp.float32),
            pltpu.VMEM((ch, D_T), jnp.float32),
            pltpu.SemaphoreType.DMA,
            pltpu.SemaphoreType.DMA,
        ],
    )
    def k(t_hbm, i_hbm, out_hbm, idx_all, r0, r1, sem0, sem1):
        wid = lax.axis_index("s") * 2 + lax.axis_index("c")
        base = wid * per_w
        pltpu.sync_copy(i_hbm.at[wid], idx_all)
        pltpu.async_copy(t_hbm.at[idx_all.at[0]], r0, sem0)
        pltpu.async_copy(t_hbm.at[idx_all.at[1]], r1, sem1)

        def pair(i, _):
            k0 = 2 * i
            pltpu.make_async_copy(t_hbm.at[idx_all.at[k0]], r0, sem0).wait()
            pltpu.sync_copy(r0, out_hbm.at[pl.ds(base + k0 * ch, ch)])

            @pl.when(k0 + 2 < nch)
            def _():
                pltpu.async_copy(t_hbm.at[idx_all.at[k0 + 2]], r0, sem0)

            pltpu.make_async_copy(t_hbm.at[idx_all.at[k0 + 1]], r1, sem1).wait()
            pltpu.sync_copy(r1, out_hbm.at[pl.ds(base + (k0 + 1) * ch, ch)])

            @pl.when(k0 + 3 < nch)
            def _():
                pltpu.async_copy(t_hbm.at[idx_all.at[k0 + 3]], r1, sem1)

            return 0

        lax.fori_loop(0, nch // 2, pair, 0)
        if nch % 2:
            kl = nch - 1
            pltpu.make_async_copy(t_hbm.at[idx_all.at[kl]], r0, sem0).wait()
            pltpu.sync_copy(r0, out_hbm.at[pl.ds(base + kl * ch, ch)])

    return k(table, idx3)


def _sc_scatter(msg, idx3, zrows):
    _, nch, ch = idx3.shape
    per_sub = nch * ch

    @functools.partial(
        pl.kernel, mesh=_sc_mesh(),
        out_type=jax.ShapeDtypeStruct((N_PAD, D_M), jnp.float32),
        scratch_types=[
            pltpu.VMEM((nch, ch), jnp.int32),
            pltpu.VMEM((ch, 128), jnp.float32),
            pltpu.VMEM((ch, 128), jnp.float32),
            pltpu.VMEM_SHARED((N_PAD, 128), jnp.float32),
            pltpu.SemaphoreType.DMA,
            pltpu.SemaphoreType.DMA,
            pltpu.SemaphoreType.DMA,
            pltpu.SemaphoreType.DMA,
        ],
    )
    def k(m_hbm, i_hbm, z_hbm, out_hbm, idx_all, m0, m1, acc,
          sl0, sl1, sa0, sa1):
        c = lax.axis_index("c")
        s = lax.axis_index("s")
        rows = N_PAD // 16
        rbase = s * rows
        base = s * per_sub
        pltpu.sync_copy(i_hbm.at[s], idx_all)

        def mrow(kk):
            return m_hbm.at[pl.ds(base + kk * ch, ch)]

        for j in range(2):
            coff = (c * 2 + j) * 128
            pltpu.sync_copy(z_hbm, acc.at[pl.ds(rbase, rows)])
            pltpu.async_copy(mrow(0).at[:, pl.ds(coff, 128)], m0, sl0)
            pltpu.async_copy(mrow(1).at[:, pl.ds(coff, 128)], m1, sl1)
            plsc.subcore_barrier()

            def pair(i, _):
                k0 = 2 * i
                pltpu.make_async_copy(mrow(k0).at[:, pl.ds(coff, 128)], m0, sl0).wait()
                pltpu.async_copy(m0, acc.at[idx_all.at[k0]], sa0, add=True)
                pltpu.make_async_copy(mrow(k0 + 1).at[:, pl.ds(coff, 128)], m1, sl1).wait()
                pltpu.async_copy(m1, acc.at[idx_all.at[k0 + 1]], sa1, add=True)
                pltpu.make_async_copy(m0, acc.at[idx_all.at[k0]], sa0).wait()

                @pl.when(k0 + 2 < nch)
                def _():
                    pltpu.async_copy(mrow(k0 + 2).at[:, pl.ds(coff, 128)], m0, sl0)

                pltpu.make_async_copy(m1, acc.at[idx_all.at[k0 + 1]], sa1).wait()

                @pl.when(k0 + 3 < nch)
                def _():
                    pltpu.async_copy(mrow(k0 + 3).at[:, pl.ds(coff, 128)], m1, sl1)

                return 0

            lax.fori_loop(0, nch // 2, pair, 0)
            if nch % 2:
                kl = nch - 1
                pltpu.make_async_copy(mrow(kl).at[:, pl.ds(coff, 128)], m0, sl0).wait()
                pltpu.sync_copy(m0, acc.at[idx_all.at[kl]], add=True)
            plsc.subcore_barrier()
            pltpu.sync_copy(acc.at[pl.ds(rbase, rows)],
                            out_hbm.at[pl.ds(rbase, rows), pl.ds(coff, 128)])
            plsc.subcore_barrier()

    return k(msg, idx3, zrows)


def _full(shape):
    n = len(shape)
    return pl.BlockSpec(shape, lambda i, _n=n: (0,) * _n)


def kernel(node_h, node_vec, edge_index, edge_attr, edge_vector, params):
    p = params
    f32 = jnp.float32
    row = edge_index[0]
    col = edge_index[1]
    ea_t = edge_attr.T
    ev_t = edge_vector.T
    nvx = node_vec[:, :, 0]
    nvy = node_vec[:, :, 1]
    nvz = node_vec[:, :, 2]

    offs = np.linspace(0.0, R_MAX, NUM_GAUSS, dtype=np.float32).reshape(1, NUM_GAUSS)
    w1 = p['mm_egvp_lv1'] @ p['vexp_W'][:, 0]
    w2 = (p['mm_egvp_lv2'] @ w1).reshape(1, EC)
    w1a = jnp.abs(w1).reshape(1, EC)

    def gv_w(pre):
        lv1t = p[pre + '_lv1'].T
        lst = p[pre + '_ls'].T
        wv = jnp.concatenate([lv1t, lv1t @ p[pre + '_lv2'].T], axis=1)
        ws = jnp.concatenate([lst, lst @ p[pre + '_gw'].T], axis=1)
        return (wv, ws, p[pre + '_gb'].reshape(1, -1))

    node_in_specs = [
        pl.BlockSpec((NB, S), lambda i: (i, 0)),
        pl.BlockSpec((NB, V), lambda i: (i, 0)),
        pl.BlockSpec((NB, V), lambda i: (i, 0)),
        pl.BlockSpec((NB, V), lambda i: (i, 0)),
    ]
    nw = gv_w('mm_node')
    T = pl.pallas_call(
        _node_table_body,
        grid=(N_NODES // NB,),
        in_specs=node_in_specs + [_full(w.shape) for w in nw],
        out_specs=pl.BlockSpec((NB, D_T), lambda i: (i, 0)),
        out_shape=jax.ShapeDtypeStruct((N_NODES, D_T), f32),
    )(node_h, nvx, nvy, nvz, *nw)

    e_lst = p['mm_egvp_ls'].T
    e_cmb = jnp.concatenate([e_lst, e_lst @ p['mm_egvp_gw'].T], axis=1)
    se_cmb = jnp.concatenate([p['mm_sca_W'].T, p['mm_e2n_W'].T], axis=1)
    ew = (w1a, w2, offs,
          e_cmb, p['mm_egvp_gb'].reshape(1, EC),
          p['mm_egvp_act_W'].T,
          se_cmb, p['mm_sca_b'].reshape(1, S),
          p['mm_e2n_b'].reshape(1, V),
          p['mm_n2e_W'].T, p['mm_n2e_b'].reshape(1, V),
          p['mm_evn_W'].T) + gv_w('mm_out')
    zrows = jnp.zeros((N_PAD // 16, 128), f32)
    npad = E_PAD2 - N_EDGES
    col_p = jnp.concatenate([col, jnp.zeros((npad,), jnp.int32)])
    row_p = jnp.concatenate([row, jnp.zeros((npad,), jnp.int32)])
    ea_tp = jnp.pad(ea_t, ((0, 0), (0, npad)))
    ev_tp = jnp.pad(ev_t, ((0, 0), (0, npad)))
    aggs = []
    e0 = 0
    for pn in PARTS:
        sl = slice(e0, e0 + pn)
        off = e0 // EB
        body = _edge_mlp_body if e0 + pn <= N_EDGES else \
            functools.partial(_edge_mlp_body, base=e0)
        e0 += pn
        ch_g = _pick_ch(pn // NW, 128, 16)
        ch_s = _pick_ch(pn // 16, 128, 8)
        col3 = col_p[sl].reshape(NW, (pn // NW) // ch_g, ch_g)
        row3 = row_p[sl].reshape(16, (pn // 16) // ch_s, ch_s)
        G = _sc_gather(T, col3)
        M = pl.pallas_call(
            body,
            grid=(pn // EB,),
            in_specs=[
                pl.BlockSpec((EB, D_T), lambda i: (i, 0)),
                pl.BlockSpec((NET, EB), lambda i, o=off: (0, i + o)),
                pl.BlockSpec((3, EB), lambda i, o=off: (0, i + o)),
            ] + [_full(w.shape) for w in ew],
            out_specs=pl.BlockSpec((EB, D_M), lambda i: (i, 0)),
            out_shape=jax.ShapeDtypeStruct((pn, D_M), f32),
        )(G, ea_tp, ev_tp, *ew)
        aggs.append(_sc_scatter(M, row3, zrows))

    cw = gv_w('cent')
    ow = gv_w('out')
    pw = cw + (p['ln_sca_g'].reshape(1, S), p['ln_sca_b'].reshape(1, S),
               p['ln_vec_g'].T, p['ln_vec_b'].T, p['act_vec_W'].T) + ow
    fs, fv = pl.pallas_call(
        _node_post_body,
        grid=(N_NODES // NB,),
        in_specs=node_in_specs
        + [pl.BlockSpec((NB, D_M), lambda i: (i, 0)) for _ in PARTS]
        + [_full(w.shape) for w in pw],
        out_specs=[pl.BlockSpec((NB, S), lambda i: (i, 0)),
                   pl.BlockSpec((NB, 3 * V), lambda i: (i, 0))],
        out_shape=[jax.ShapeDtypeStruct((N_NODES, S), f32),
                   jax.ShapeDtypeStruct((N_NODES, 3 * V), f32)],
    )(node_h, nvx, nvy, nvz, *aggs, *pw)

    out_vec = jnp.swapaxes(fv.reshape(N_NODES, 3, V), 1, 2)
    return fs, out_vec


# speedup vs baseline: 1.0602x; 1.0602x over previous
import functools

import jax
import jax.numpy as jnp
import numpy as np
from jax import lax
from jax.experimental import pallas as pl
from jax.experimental.pallas import tpu as pltpu
from jax.experimental.pallas import tpu_sc as plsc

N_NODES = 10000
N_EDGES = 160000
S = 256
V = 64
EC = 64
NET = 4
R_MAX = 10.0
NUM_GAUSS = EC - NET
D_T = 256
D_M = 512

N_PAD = 10240
NW = 32
CH = 128
NB = 1000
EB = 640
PARTS = (40960, 40960, 40960, 37120)

_SLOPE = 0.2


def _leaky(x):
    return jnp.where(x >= 0, x, 0.01 * x)


def _gv_planes(sca, vx, vy, vz, wv, ws, gb, outs):
    f32 = jnp.float32
    B = sca.shape[0]
    vs = jnp.concatenate([vx, vy, vz], axis=0)
    ys = jnp.dot(vs, wv, preferred_element_type=f32)
    vi = [ys[c * B:(c + 1) * B, :64] for c in range(3)]
    ovp = [ys[c * B:(c + 1) * B, 64:] for c in range(3)]
    vnorm = jnp.sqrt(vi[0] * vi[0] + vi[1] * vi[1] + vi[2] * vi[2])
    sc = jnp.concatenate([vnorm, sca], axis=-1)
    so = jnp.dot(sc, ws, preferred_element_type=f32)
    out_s = so[:, :outs]
    gate = jax.nn.sigmoid(so[:, outs:] + gb)
    ov = [gate * v for v in ovp]
    return out_s, ov


def _node_table_body(nh, nvx, nvy, nvz, wv, ws, gb, out):
    f32 = jnp.float32
    i32 = jnp.int32
    ns, nv = _gv_planes(nh[...], nvx[...], nvy[...], nvz[...],
                        wv[...], ws[...], gb[...], S)
    z = jnp.zeros((ns.shape[0], 64), f32)
    hi = jnp.concatenate([nv[0], nv[1], nv[2], z], axis=-1)
    lob = jax.lax.bitcast_convert_type(ns.astype(jnp.bfloat16).astype(f32), i32)
    hib = jax.lax.bitcast_convert_type(hi.astype(jnp.bfloat16).astype(f32), i32)
    packed = jax.lax.shift_right_logical(lob, 16) | (hib & i32(-65536))
    out[...] = jax.lax.bitcast_convert_type(packed, f32)


def _node_post_body(nh, nvx, nvy, nvz, agg0, agg1, agg2, agg3,
                    c_wv, c_ws, c_gb,
                    lnsg, lnsb, lnvg, lnvb, actwt,
                    o_wv, o_ws, o_gb,
                    out_s, out_v):
    f32 = jnp.float32
    cs, cv = _gv_planes(nh[...], nvx[...], nvy[...], nvz[...],
                        c_wv[...], c_ws[...], c_gb[...], S)
    a = (agg0[...] + agg1[...]) + (agg2[...] + agg3[...])
    osca = cs + a[:, :S]
    ov = [cv[c] + a[:, S + 64 * c:S + 64 * (c + 1)] for c in range(3)]
    mu = jnp.mean(osca, axis=-1, keepdims=True)
    var = jnp.mean(osca * osca, axis=-1, keepdims=True) - mu * mu
    osca = (osca - mu) * jax.lax.rsqrt(var + 1e-5) * lnsg[...] + lnsb[...]
    sv = ov[0] + ov[1] + ov[2]
    mu_v = jnp.sum(sv, axis=-1, keepdims=True) * (1.0 / 192.0)
    sq = ov[0] * ov[0] + ov[1] * ov[1] + ov[2] * ov[2]
    var_v = jnp.sum(sq, axis=-1, keepdims=True) * (1.0 / 192.0) - mu_v * mu_v
    inv = jax.lax.rsqrt(var_v + 1e-5)
    g = lnvg[...]
    b = lnvb[...]
    ov = [(ov[c] - mu_v) * inv * g[c:c + 1, :] + b[c:c + 1, :] for c in range(3)]
    s_act = _leaky(osca)
    dv = [jnp.dot(v, actwt[...], preferred_element_type=f32) for v in ov]
    dot = ov[0] * dv[0] + ov[1] * dv[1] + ov[2] * dv[2]
    d2 = dv[0] * dv[0] + dv[1] * dv[1] + dv[2] * dv[2]
    msk = (dot >= 0).astype(f32)
    ratio = (1.0 - msk) * (dot / (d2 + 1e-6))
    v_act = [_SLOPE * ov[c] + (1.0 - _SLOPE) * (ov[c] - ratio * dv[c])
             for c in range(3)]
    fs, fv = _gv_planes(s_act, v_act[0], v_act[1], v_act[2],
                        o_wv[...], o_ws[...], o_gb[...], S)
    out_s[...] = fs
    out_v[...] = jnp.concatenate([fv[0], fv[1], fv[2]], axis=-1)


def _edge_mlp_body(g, ea_t, ev_t,
                   w1a, w2, offs, e_cmb, e_gb, actwt,
                   se_cmb, scab, e2nb, n2ewt, n2eb, evnwt,
                   m_wv, m_ws, m_gb,
                   out, *, base=None):
    f32 = jnp.float32
    i32 = jnp.int32
    gi = jax.lax.bitcast_convert_type(g[...], i32)
    ns_j = jax.lax.bitcast_convert_type(jax.lax.shift_left(gi, 16), f32)
    hiv = jax.lax.bitcast_convert_type(gi & i32(-65536), f32)
    eye = (jax.lax.broadcasted_iota(jnp.int32, (EB, EB), 0)
           == jax.lax.broadcasted_iota(jnp.int32, (EB, EB), 1)).astype(f32)
    tdn = (((1,), (1,)), ((), ()))
    evec = jax.lax.dot_general(eye, ev_t[...], tdn,
                               preferred_element_type=f32)
    eattr = jax.lax.dot_general(eye, ea_t[...], tdn,
                                preferred_element_type=f32)
    d2r = jnp.sum(evec * evec, axis=-1, keepdims=True)
    d = jnp.sqrt(d2r)
    diff = d - offs[...]
    step = R_MAX / (NUM_GAUSS - 1)
    coeff = -0.5 / (step * step)
    gexp = jnp.exp(coeff * diff * diff)
    esf = jnp.concatenate([gexp, eattr], axis=-1)
    inv_d = 1.0 / (d + 1e-7)
    evn = evec * inv_d
    q = d2r * inv_d * inv_d
    vnorm_e = jnp.sqrt(q) * jnp.abs(w1a[...])
    sc = jnp.concatenate([vnorm_e, esf], axis=-1)
    er = jnp.dot(sc, e_cmb[...], preferred_element_type=f32)
    es0 = er[:, :EC]
    gate_e = jax.nn.sigmoid(er[:, EC:] + e_gb[...])
    a = gate_e * w2[...]
    A = jnp.dot(a, actwt[...], preferred_element_type=f32)
    dot = a * A * q
    dd = A * A * q
    msk = (dot >= 0).astype(f32)
    a2 = _SLOPE * a + (1.0 - _SLOPE) * (a - (1.0 - msk) * (dot / (dd + 1e-6)) * A)
    es = _leaky(es0)
    ser = jnp.dot(es, se_cmb[...], preferred_element_type=f32)
    y_sca = ns_j * (ser[:, :S] + scab[...])
    alpha = ser[:, S:] + e2nb[...]
    beta = (jnp.dot(ns_j, n2ewt[...], preferred_element_type=f32) + n2eb[...]) * \
        jnp.dot(a2, evnwt[...], preferred_element_type=f32)
    yv = [alpha * hiv[:, 64 * c:64 * (c + 1)] + beta * evn[:, c:c + 1]
          for c in range(3)]
    m_sca, m_vec = _gv_planes(y_sca, yv[0], yv[1], yv[2],
                              m_wv[...], m_ws[...], m_gb[...], S)
    C = 0.5 * (jnp.cos(d * (np.pi / R_MAX)) + 1.0)
    C = C * (d <= R_MAX).astype(f32) * (d >= 0.0).astype(f32)
    if base is not None:
        eid = base + pl.program_id(0) * EB + \
            jax.lax.broadcasted_iota(jnp.int32, (EB, 1), 0)
        C = C * (eid < N_EDGES).astype(f32)
    z = jnp.zeros((EB, 64), f32)
    out[...] = jnp.concatenate(
        [m_sca * C, m_vec[0] * C, m_vec[1] * C, m_vec[2] * C, z], axis=-1)


def _sc_mesh():
    return plsc.VectorSubcoreMesh(core_axis_name="c", subcore_axis_name="s",
                                  num_cores=2)


def _pick_ch(per, cap, align):
    for d in range(cap - cap % align, align - 1, -align):
        if per % d == 0:
            return d
    raise ValueError(per)


def _sc_gather(table, idx3):
    nw, nch, ch = idx3.shape
    per_w = nch * ch
    n = NW * per_w

    @functools.partial(
        pl.kernel, mesh=_sc_mesh(),
        out_type=jax.ShapeDtypeStruct((n, D_T), jnp.float32),
        scratch_types=[
            pltpu.VMEM((nch, ch), jnp.int32),
            pltpu.VMEM((ch, D_T), jnp.float32),
            pltpu.VMEM((ch, D_T), jnp.float32),
            pltpu.SemaphoreType.DMA,
            pltpu.SemaphoreType.DMA,
        ],
    )
    def k(t_hbm, i_hbm, out_hbm, idx_all, r0, r1, sem0, sem1):
        wid = lax.axis_index("s") * 2 + lax.axis_index("c")
        base = wid * per_w
        pltpu.sync_copy(i_hbm.at[wid], idx_all)
        pltpu.async_copy(t_hbm.at[idx_all.at[0]], r0, sem0)
        pltpu.async_copy(t_hbm.at[idx_all.at[1]], r1, sem1)

        def pair(i, _):
            k0 = 2 * i
            pltpu.make_async_copy(t_hbm.at[idx_all.at[k0]], r0, sem0).wait()
            pltpu.sync_copy(r0, out_hbm.at[pl.ds(base + k0 * ch, ch)])

            @pl.when(k0 + 2 < nch)
            def _():
                pltpu.async_copy(t_hbm.at[idx_all.at[k0 + 2]], r0, sem0)

            pltpu.make_async_copy(t_hbm.at[idx_all.at[k0 + 1]], r1, sem1).wait()
            pltpu.sync_copy(r1, out_hbm.at[pl.ds(base + (k0 + 1) * ch, ch)])

            @pl.when(k0 + 3 < nch)
            def _():
                pltpu.async_copy(t_hbm.at[idx_all.at[k0 + 3]], r1, sem1)

            return 0

        lax.fori_loop(0, nch // 2, pair, 0)
        if nch % 2:
            kl = nch - 1
            pltpu.make_async_copy(t_hbm.at[idx_all.at[kl]], r0, sem0).wait()
            pltpu.sync_copy(r0, out_hbm.at[pl.ds(base + kl * ch, ch)])

    return k(table, idx3)


def _sc_scatter(msg, idx3, zrows):
    _, nch, ch = idx3.shape
    per_sub = nch * ch

    @functools.partial(
        pl.kernel, mesh=_sc_mesh(),
        out_type=jax.ShapeDtypeStruct((N_PAD, D_M), jnp.float32),
        scratch_types=[
            pltpu.VMEM((nch, ch), jnp.int32),
            pltpu.VMEM((ch, 128), jnp.float32),
            pltpu.VMEM((ch, 128), jnp.float32),
            pltpu.VMEM_SHARED((N_PAD, 128), jnp.float32),
            pltpu.SemaphoreType.DMA,
            pltpu.SemaphoreType.DMA,
            pltpu.SemaphoreType.DMA,
            pltpu.SemaphoreType.DMA,
        ],
    )
    def k(m_hbm, i_hbm, z_hbm, out_hbm, idx_all, m0, m1, acc,
          sl0, sl1, sa0, sa1):
        c = lax.axis_index("c")
        s = lax.axis_index("s")
        rows = N_PAD // 16
        rbase = s * rows
        base = s * per_sub
        pltpu.sync_copy(i_hbm.at[s], idx_all)

        def mrow(kk):
            return m_hbm.at[pl.ds(base + kk * ch, ch)]

        for j in range(2):
            coff = (c * 2 + j) * 128
            pltpu.sync_copy(z_hbm, acc.at[pl.ds(rbase, rows)])
            pltpu.async_copy(mrow(0).at[:, pl.ds(coff, 128)], m0, sl0)
            pltpu.async_copy(mrow(1).at[:, pl.ds(coff, 128)], m1, sl1)
            plsc.subcore_barrier()

            def pair(i, _):
                k0 = 2 * i
                pltpu.make_async_copy(mrow(k0).at[:, pl.ds(coff, 128)], m0, sl0).wait()
                pltpu.async_copy(m0, acc.at[idx_all.at[k0]], sa0, add=True)
                pltpu.make_async_copy(mrow(k0 + 1).at[:, pl.ds(coff, 128)], m1, sl1).wait()
                pltpu.async_copy(m1, acc.at[idx_all.at[k0 + 1]], sa1, add=True)
                pltpu.make_async_copy(m0, acc.at[idx_all.at[k0]], sa0).wait()

                @pl.when(k0 + 2 < nch)
                def _():
                    pltpu.async_copy(mrow(k0 + 2).at[:, pl.ds(coff, 128)], m0, sl0)

                pltpu.make_async_copy(m1, acc.at[idx_all.at[k0 + 1]], sa1).wait()

                @pl.when(k0 + 3 < nch)
                def _():
                    pltpu.async_copy(mrow(k0 + 3).at[:, pl.ds(coff, 128)], m1, sl1)

                return 0

            lax.fori_loop(0, nch // 2, pair, 0)
            if nch % 2:
                kl = nch - 1
                pltpu.make_async_copy(mrow(kl).at[:, pl.ds(coff, 128)], m0, sl0).wait()
                pltpu.sync_copy(m0, acc.at[idx_all.at[kl]], add=True)
            plsc.subcore_barrier()
            pltpu.sync_copy(acc.at[pl.ds(rbase, rows)],
                            out_hbm.at[pl.ds(rbase, rows), pl.ds(coff, 128)])
            plsc.subcore_barrier()

    return k(msg, idx3, zrows)


def _full(shape):
    n = len(shape)
    return pl.BlockSpec(shape, lambda i, _n=n: (0,) * _n)


def kernel(node_h, node_vec, edge_index, edge_attr, edge_vector, params):
    p = params
    f32 = jnp.float32
    row = edge_index[0]
    col = edge_index[1]
    ea_t = edge_attr.T
    ev_t = edge_vector.T
    nvx = node_vec[:, :, 0]
    nvy = node_vec[:, :, 1]
    nvz = node_vec[:, :, 2]

    offs = np.linspace(0.0, R_MAX, NUM_GAUSS, dtype=np.float32).reshape(1, NUM_GAUSS)
    w1 = p['mm_egvp_lv1'] @ p['vexp_W'][:, 0]
    w2 = (p['mm_egvp_lv2'] @ w1).reshape(1, EC)
    w1a = jnp.abs(w1).reshape(1, EC)

    def gv_w(pre):
        lv1t = p[pre + '_lv1'].T
        lst = p[pre + '_ls'].T
        wv = jnp.concatenate([lv1t, lv1t @ p[pre + '_lv2'].T], axis=1)
        ws = jnp.concatenate([lst, lst @ p[pre + '_gw'].T], axis=1)
        return (wv, ws, p[pre + '_gb'].reshape(1, -1))

    node_in_specs = [
        pl.BlockSpec((NB, S), lambda i: (i, 0)),
        pl.BlockSpec((NB, V), lambda i: (i, 0)),
        pl.BlockSpec((NB, V), lambda i: (i, 0)),
        pl.BlockSpec((NB, V), lambda i: (i, 0)),
    ]
    nw = gv_w('mm_node')
    T = pl.pallas_call(
        _node_table_body,
        grid=(N_NODES // NB,),
        in_specs=node_in_specs + [_full(w.shape) for w in nw],
        out_specs=pl.BlockSpec((NB, D_T), lambda i: (i, 0)),
        out_shape=jax.ShapeDtypeStruct((N_NODES, D_T), f32),
    )(node_h, nvx, nvy, nvz, *nw)

    e_lst = p['mm_egvp_ls'].T
    e_cmb = jnp.concatenate([e_lst, e_lst @ p['mm_egvp_gw'].T], axis=1)
    se_cmb = jnp.concatenate([p['mm_sca_W'].T, p['mm_e2n_W'].T], axis=1)
    ew = (w1a, w2, offs,
          e_cmb, p['mm_egvp_gb'].reshape(1, EC),
          p['mm_egvp_act_W'].T,
          se_cmb, p['mm_sca_b'].reshape(1, S),
          p['mm_e2n_b'].reshape(1, V),
          p['mm_n2e_W'].T, p['mm_n2e_b'].reshape(1, V),
          p['mm_evn_W'].T) + gv_w('mm_out')
    zrows = jnp.zeros((N_PAD // 16, 128), f32)
    aggs = []
    e0 = 0
    for pn in PARTS:
        sl = slice(e0, e0 + pn)
        off = e0 // EB
        body = _edge_mlp_body
        e0 += pn
        ch_g = _pick_ch(pn // NW, 128, 8)
        ch_s = _pick_ch(pn // 16, 128, 8)
        col3 = col[sl].reshape(NW, (pn // NW) // ch_g, ch_g)
        row3 = row[sl].reshape(16, (pn // 16) // ch_s, ch_s)
        G = _sc_gather(T, col3)
        M = pl.pallas_call(
            body,
            grid=(pn // EB,),
            in_specs=[
                pl.BlockSpec((EB, D_T), lambda i: (i, 0)),
                pl.BlockSpec((NET, EB), lambda i, o=off: (0, i + o)),
                pl.BlockSpec((3, EB), lambda i, o=off: (0, i + o)),
            ] + [_full(w.shape) for w in ew],
            out_specs=pl.BlockSpec((EB, D_M), lambda i: (i, 0)),
            out_shape=jax.ShapeDtypeStruct((pn, D_M), f32),
        )(G, ea_t, ev_t, *ew)
        aggs.append(_sc_scatter(M, row3, zrows))

    cw = gv_w('cent')
    ow = gv_w('out')
    pw = cw + (p['ln_sca_g'].reshape(1, S), p['ln_sca_b'].reshape(1, S),
               p['ln_vec_g'].T, p['ln_vec_b'].T, p['act_vec_W'].T) + ow
    fs, fv = pl.pallas_call(
        _node_post_body,
        grid=(N_NODES // NB,),
        in_specs=node_in_specs
        + [pl.BlockSpec((NB, D_M), lambda i: (i, 0)) for _ in PARTS]
        + [_full(w.shape) for w in pw],
        out_specs=[pl.BlockSpec((NB, S), lambda i: (i, 0)),
                   pl.BlockSpec((NB, 3 * V), lambda i: (i, 0))],
        out_shape=[jax.ShapeDtypeStruct((N_NODES, S), f32),
                   jax.ShapeDtypeStruct((N_NODES, 3 * V), f32)],
    )(node_h, nvx, nvy, nvz, *aggs, *pw)

    out_vec = jnp.swapaxes(fv.reshape(N_NODES, 3, V), 1, 2)
    return fs, out_vec
